# baseline (speedup 1.0000x reference)
import functools

import jax
import jax.numpy as jnp
from jax import lax
from jax.experimental import pallas as pl
from jax.experimental.pallas import tpu as pltpu
from jax.experimental.pallas import tpu_sc as plsc

_NC = 2
_NS = 16
_NW = _NC * _NS


def _split(total, nw=_NW):
    ch = ((total + nw - 1) // nw + 7) // 8 * 8
    last = total - (nw - 1) * ch
    assert last > 0 and last % 8 == 0 and ch % 8 == 0
    return ch, last


def _score_grid_body(f_ref, w_ref, a_ref, b_ref, s_ref):
    g = jnp.dot(a_ref[...], w_ref[...],
                preferred_element_type=jnp.float32)
    c0 = jnp.sum(a_ref[...] * b_ref[...])
    f = f_ref[0]
    s_ref[0, 0] = jnp.sum(f * g[0][:, None, None], axis=0) + c0


def _score_grid(feat, W, a, b, HB):
    Bn, C, H, Wg = feat.shape
    CE = W.shape[0]
    return pl.pallas_call(
        _score_grid_body,
        grid=(Bn, H // HB),
        in_specs=[
            pl.BlockSpec((1, C, HB, Wg), lambda bi, i: (bi, 0, i, 0)),
            pl.BlockSpec((CE, C), lambda bi, i: (0, 0)),
            pl.BlockSpec((1, CE), lambda bi, i: (0, 0)),
            pl.BlockSpec((1, CE), lambda bi, i: (0, 0)),
        ],
        out_specs=pl.BlockSpec((1, 1, HB, Wg), lambda bi, i: (bi, 0, i, 0)),
        out_shape=jax.ShapeDtypeStruct((Bn, 1, H, Wg), jnp.float32),
        compiler_params=pltpu.CompilerParams(
            dimension_semantics=("parallel", "parallel")),
    )(feat, W, a.reshape(1, CE), b.reshape(1, CE))


def _point_body(p_ref, sv_ref, sr_ref, w_ref, b_ref, a_ref,
                xp_ref, av_ref, ar_ref):
    emb = lax.dot_general(w_ref[...], p_ref[0],
                          (((1,), (1,)), ((), ())),
                          preferred_element_type=jnp.float32) + b_ref[...]
    sp = jnp.sum(emb * a_ref[...], axis=0, keepdims=True)
    sv = sv_ref[0]
    sr = sr_ref[0]
    m = jnp.maximum(sp, jnp.maximum(sv, sr))
    ep = jnp.exp(sp - m)
    ev = jnp.exp(sv - m)
    er = jnp.exp(sr - m)
    inv = 1.0 / (ep + ev + er)
    xp_ref[0] = emb * (ep * inv)
    av_ref[0] = ev * inv
    ar_ref[0] = er * inv


def _point_kernel(pT, sv_pt, sr_pt, Wp, bp, ap, T):
    Bn, N, C = pT.shape
    CE = Wp.shape[0]
    nt = pl.cdiv(N, T)
    return pl.pallas_call(
        _point_body,
        grid=(Bn, nt),
        in_specs=[
            pl.BlockSpec((1, T, C), lambda bi, i: (bi, i, 0)),
            pl.BlockSpec((1, 1, T), lambda bi, i: (bi, 0, i)),
            pl.BlockSpec((1, 1, T), lambda bi, i: (bi, 0, i)),
            pl.BlockSpec((CE, C), lambda bi, i: (0, 0)),
            pl.BlockSpec((CE, 1), lambda bi, i: (0, 0)),
            pl.BlockSpec((CE, 1), lambda bi, i: (0, 0)),
        ],
        out_specs=[
            pl.BlockSpec((1, CE, T), lambda bi, i: (bi, 0, i)),
            pl.BlockSpec((1, 1, T), lambda bi, i: (bi, 0, i)),
            pl.BlockSpec((1, 1, T), lambda bi, i: (bi, 0, i)),
        ],
        out_shape=[
            jax.ShapeDtypeStruct((Bn, CE, N), jnp.float32),
            jax.ShapeDtypeStruct((Bn, 1, N), jnp.float32),
            jax.ShapeDtypeStruct((Bn, 1, N), jnp.float32),
        ],
        compiler_params=pltpu.CompilerParams(
            dimension_semantics=("parallel", "parallel")),
    )(pT, sv_pt, sr_pt, Wp, bp.reshape(CE, 1), ap.reshape(CE, 1))


def _map_body(hb, f_ref, w_ref, b_ref, wt_ref, o_ref):
    for h in range(hb):
        emb = jnp.dot(w_ref[...], f_ref[0, :, h, :],
                      preferred_element_type=jnp.float32) + b_ref[...]
        o_ref[0, :, h, :] = emb * wt_ref[0, h, :][None, :]


def _map_kernel(feat, W, b, wt, HB):
    Bn, C, H, Wg = feat.shape
    CE = W.shape[0]
    return pl.pallas_call(
        functools.partial(_map_body, HB),
        grid=(Bn, H // HB),
        in_specs=[
            pl.BlockSpec((1, C, HB, Wg), lambda bi, i: (bi, 0, i, 0)),
            pl.BlockSpec((CE, C), lambda bi, i: (0, 0)),
            pl.BlockSpec((CE, 1), lambda bi, i: (0, 0)),
            pl.BlockSpec((1, HB, Wg), lambda bi, i: (bi, i, 0)),
        ],
        out_specs=pl.BlockSpec((1, CE, HB, Wg), lambda bi, i: (bi, 0, i, 0)),
        out_shape=jax.ShapeDtypeStruct((Bn, CE, H, Wg), jnp.float32),
        compiler_params=pltpu.CompilerParams(
            dimension_semantics=("parallel", "parallel")),
    )(feat, W, b.reshape(CE, 1), wt)


def _sc_gather(tab, lin):
    PT = lin.shape[0]
    CH, CHL = _split(PT)
    mesh = plsc.VectorSubcoreMesh(core_axis_name="c", subcore_axis_name="s")

    @functools.partial(
        pl.kernel, mesh=mesh,
        out_type=jax.ShapeDtypeStruct((PT,), jnp.float32),
        scratch_types=[pltpu.VMEM((CH,), jnp.int32),
                       pltpu.VMEM((CH,), jnp.float32),
                       pltpu.VMEM((CHL,), jnp.int32),
                       pltpu.VMEM((CHL,), jnp.float32),
                       pltpu.SemaphoreType.DMA],
    )
    def gk(tab_h, lin_h, out_h, iv, rv, ivt, rvt, sem):
        wid = lax.axis_index("s") * _NC + lax.axis_index("c")
        base = wid * CH

        @pl.when(wid < _NW - 1)
        def _main():
            pltpu.sync_copy(lin_h.at[pl.ds(base, CH)], iv)
            pltpu.async_copy(tab_h.at[iv], rv, sem).wait()
            pltpu.sync_copy(rv, out_h.at[pl.ds(base, CH)])

        @pl.when(wid == _NW - 1)
        def _tail():
            pltpu.sync_copy(lin_h.at[pl.ds(base, CHL)], ivt)
            pltpu.async_copy(tab_h.at[ivt], rvt, sem).wait()
            pltpu.sync_copy(rvt, out_h.at[pl.ds(base, CHL)])

    return gk(tab, lin)


def _sc_scatter(avals, rvals, linv, linr, zrow, GVt, GRt):
    PT = avals.shape[0]
    CH, CHL = _split(PT, _NS)
    SGV = GVt // _NS
    SGR = GRt // _NS
    mesh = plsc.VectorSubcoreMesh(core_axis_name="c", subcore_axis_name="s")

    @functools.partial(
        pl.kernel, mesh=mesh,
        out_type=[jax.ShapeDtypeStruct((GVt,), jnp.float32),
                  jax.ShapeDtypeStruct((GRt,), jnp.float32)],
        scratch_types=[pltpu.VMEM((CH,), jnp.int32),
                       pltpu.VMEM((CH,), jnp.float32),
                       pltpu.VMEM((CHL,), jnp.int32),
                       pltpu.VMEM((CHL,), jnp.float32),
                       pltpu.VMEM_SHARED((GVt,), jnp.float32),
                       pltpu.VMEM_SHARED((GRt,), jnp.float32),
                       pltpu.SemaphoreType.DMA],
    )
    def sk(av_h, ar_h, lv_h, lr_h, z_h, ov_h, or_h,
           iv, vv, ivt, vvt, accv, accr, sem):
        cid = lax.axis_index("c")
        sid = lax.axis_index("s")
        base = sid * CH

        @pl.when(cid == 0)
        def _zv():
            pltpu.sync_copy(z_h.at[pl.ds(0, SGV)],
                            accv.at[pl.ds(sid * SGV, SGV)])

        @pl.when(cid == 1)
        def _zr():
            pltpu.sync_copy(z_h.at[pl.ds(0, SGR)],
                            accr.at[pl.ds(sid * SGR, SGR)])

        plsc.subcore_barrier()

        @pl.when((cid == 0) & (sid < _NS - 1))
        def _v_main():
            pltpu.sync_copy(lv_h.at[pl.ds(base, CH)], iv)
            pltpu.sync_copy(av_h.at[pl.ds(base, CH)], vv)
            pltpu.sync_copy(vv, accv.at[iv], add=True)

        @pl.when((cid == 0) & (sid == _NS - 1))
        def _v_tail():
            pltpu.sync_copy(lv_h.at[pl.ds(base, CHL)], ivt)
            pltpu.sync_copy(av_h.at[pl.ds(base, CHL)], vvt)
            pltpu.sync_copy(vvt, accv.at[ivt], add=True)

        @pl.when((cid == 1) & (sid < _NS - 1))
        def _r_main():
            pltpu.sync_copy(lr_h.at[pl.ds(base, CH)], iv)
            pltpu.sync_copy(ar_h.at[pl.ds(base, CH)], vv)
            pltpu.sync_copy(vv, accr.at[iv], add=True)

        @pl.when((cid == 1) & (sid == _NS - 1))
        def _r_tail():
            pltpu.sync_copy(lr_h.at[pl.ds(base, CHL)], ivt)
            pltpu.sync_copy(ar_h.at[pl.ds(base, CHL)], vvt)
            pltpu.sync_copy(vvt, accr.at[ivt], add=True)

        plsc.subcore_barrier()

        @pl.when(cid == 0)
        def _pv():
            pltpu.sync_copy(accv.at[pl.ds(sid * SGV, SGV)],
                            ov_h.at[pl.ds(sid * SGV, SGV)])

        @pl.when(cid == 1)
        def _pr():
            pltpu.sync_copy(accr.at[pl.ds(sid * SGR, SGR)],
                            or_h.at[pl.ds(sid * SGR, SGR)])

    return sk(avals, rvals, linv, linr, zrow)


def kernel(p_feat, v_feat, r_feat, v2p_ind, r2p_ind,
           Wp, bp, Wv, bv, Wr, br, ap, av, ar):
    B, CP, N = p_feat.shape
    CV, HV, WVG = v_feat.shape[1:]
    CR, HR, WRG = r_feat.shape[1:]
    CE = Wp.shape[0]
    GV = HV * WVG
    GR = HR * WRG

    sv_grid = _score_grid(v_feat, Wv, av, bv, HB=64)
    sr_grid = _score_grid(r_feat, Wr, ar, br, HB=16)
    RXC = 64
    src_compact = sr_grid[:, :, :, :RXC].reshape(-1)

    viT = jnp.transpose(v2p_ind, (0, 2, 1)).astype(jnp.int32)
    riT = jnp.transpose(r2p_ind, (0, 2, 1)).astype(jnp.int32)
    offv = (jnp.arange(B, dtype=jnp.int32) * GV)[:, None]
    offr = (jnp.arange(B, dtype=jnp.int32) * GR)[:, None]
    linv = (viT[:, 0, :] * WVG + viT[:, 1, :] + offv).reshape(-1)
    linr = (riT[:, 0, :] * WRG + riT[:, 1, :] + offr).reshape(-1)
    offrc = (jnp.arange(B, dtype=jnp.int32) * (HR * RXC))[:, None]
    linrc = (riT[:, 0, :] * RXC + riT[:, 1, :] + offrc).reshape(-1)

    sv_pt = _sc_gather(sv_grid.reshape(-1), linv).reshape(B, 1, N)
    sr_pt = _sc_gather(src_compact, linrc).reshape(B, 1, N)

    pT = jnp.transpose(p_feat, (0, 2, 1))
    xp, alpha_v, alpha_r = _point_kernel(pT, sv_pt, sr_pt, Wp, bp, ap,
                                         T=8192)

    zrow = jnp.zeros((B * GR // _NS,), jnp.float32)
    wv, wr = _sc_scatter(alpha_v.reshape(-1), alpha_r.reshape(-1),
                         linv, linr, zrow, B * GV, B * GR)

    xv = _map_kernel(v_feat, Wv, bv, wv.reshape(B, HV, WVG), HB=64)
    xr = _map_kernel(r_feat, Wr, br, wr.reshape(B, HR, WRG), HB=16)
    return (xp, xv, xr)


# speedup vs baseline: 1.0470x; 1.0470x over previous
import functools

import jax
import jax.numpy as jnp
from jax import lax
from jax.experimental import pallas as pl
from jax.experimental.pallas import tpu as pltpu
from jax.experimental.pallas import tpu_sc as plsc

_NC = 2
_NS = 16
_NW = _NC * _NS


def _split(total, nw=_NW):
    ch = ((total + nw - 1) // nw + 7) // 8 * 8
    last = total - (nw - 1) * ch
    assert last > 0 and last % 8 == 0 and ch % 8 == 0
    return ch, last


def _score_grid_body(f_ref, w_ref, a_ref, b_ref, s_ref):
    g = jnp.dot(a_ref[...], w_ref[...],
                preferred_element_type=jnp.float32)
    c0 = jnp.sum(a_ref[...] * b_ref[...])
    f = f_ref[0]
    s_ref[0, 0] = jnp.sum(f * g[0][:, None, None], axis=0) + c0


def _score_grid(feat, W, a, b, HB):
    Bn, C, H, Wg = feat.shape
    CE = W.shape[0]
    return pl.pallas_call(
        _score_grid_body,
        grid=(Bn, H // HB),
        in_specs=[
            pl.BlockSpec((1, C, HB, Wg), lambda bi, i: (bi, 0, i, 0)),
            pl.BlockSpec((CE, C), lambda bi, i: (0, 0)),
            pl.BlockSpec((1, CE), lambda bi, i: (0, 0)),
            pl.BlockSpec((1, CE), lambda bi, i: (0, 0)),
        ],
        out_specs=pl.BlockSpec((1, 1, HB, Wg), lambda bi, i: (bi, 0, i, 0)),
        out_shape=jax.ShapeDtypeStruct((Bn, 1, H, Wg), jnp.float32),
        compiler_params=pltpu.CompilerParams(
            dimension_semantics=("parallel", "parallel")),
    )(feat, W, a.reshape(1, CE), b.reshape(1, CE))


def _point_body(p_ref, sv_ref, sr_ref, w_ref, b_ref, a_ref,
                xp_ref, av_ref, ar_ref):
    emb = lax.dot_general(w_ref[...], p_ref[0],
                          (((1,), (1,)), ((), ())),
                          preferred_element_type=jnp.float32) + b_ref[...]
    sp = jnp.sum(emb * a_ref[...], axis=0, keepdims=True)
    sv = sv_ref[0]
    sr = sr_ref[0]
    m = jnp.maximum(sp, jnp.maximum(sv, sr))
    ep = jnp.exp(sp - m)
    ev = jnp.exp(sv - m)
    er = jnp.exp(sr - m)
    inv = 1.0 / (ep + ev + er)
    xp_ref[0] = emb * (ep * inv)
    av_ref[0] = ev * inv
    ar_ref[0] = er * inv


def _point_kernel(pT, sv_pt, sr_pt, Wp, bp, ap, T):
    Bn, N, C = pT.shape
    CE = Wp.shape[0]
    nt = pl.cdiv(N, T)
    return pl.pallas_call(
        _point_body,
        grid=(Bn, nt),
        in_specs=[
            pl.BlockSpec((1, T, C), lambda bi, i: (bi, i, 0)),
            pl.BlockSpec((1, 1, T), lambda bi, i: (bi, 0, i)),
            pl.BlockSpec((1, 1, T), lambda bi, i: (bi, 0, i)),
            pl.BlockSpec((CE, C), lambda bi, i: (0, 0)),
            pl.BlockSpec((CE, 1), lambda bi, i: (0, 0)),
            pl.BlockSpec((CE, 1), lambda bi, i: (0, 0)),
        ],
        out_specs=[
            pl.BlockSpec((1, CE, T), lambda bi, i: (bi, 0, i)),
            pl.BlockSpec((1, 1, T), lambda bi, i: (bi, 0, i)),
            pl.BlockSpec((1, 1, T), lambda bi, i: (bi, 0, i)),
        ],
        out_shape=[
            jax.ShapeDtypeStruct((Bn, CE, N), jnp.float32),
            jax.ShapeDtypeStruct((Bn, 1, N), jnp.float32),
            jax.ShapeDtypeStruct((Bn, 1, N), jnp.float32),
        ],
        compiler_params=pltpu.CompilerParams(
            dimension_semantics=("parallel", "parallel")),
    )(pT, sv_pt, sr_pt, Wp, bp.reshape(CE, 1), ap.reshape(CE, 1))


def _map_body(hb, wg, f_ref, w_ref, b_ref, wt_ref, o_ref):
    for h in range(hb):
        emb = jnp.dot(w_ref[...], f_ref[0, :, h, :],
                      preferred_element_type=jnp.float32) + b_ref[...]
        o_ref[0, :, h, :] = emb * wt_ref[pl.ds(h * wg, wg)][None, :]


def _map_kernel(feat, W, b, wt, HB):
    Bn, C, H, Wg = feat.shape
    CE = W.shape[0]
    nh = H // HB
    return pl.pallas_call(
        functools.partial(_map_body, HB, Wg),
        grid=(Bn, nh),
        in_specs=[
            pl.BlockSpec((1, C, HB, Wg), lambda bi, i: (bi, 0, i, 0)),
            pl.BlockSpec((CE, C), lambda bi, i: (0, 0)),
            pl.BlockSpec((CE, 1), lambda bi, i: (0, 0)),
            pl.BlockSpec((HB * Wg,), lambda bi, i, nh=nh: (bi * nh + i,)),
        ],
        out_specs=pl.BlockSpec((1, CE, HB, Wg), lambda bi, i: (bi, 0, i, 0)),
        out_shape=jax.ShapeDtypeStruct((Bn, CE, H, Wg), jnp.float32),
        compiler_params=pltpu.CompilerParams(
            dimension_semantics=("parallel", "parallel")),
    )(feat, W, b.reshape(CE, 1), wt)


def _sc_gather(tab, lin):
    PT = lin.shape[0]
    CH, CHL = _split(PT)
    mesh = plsc.VectorSubcoreMesh(core_axis_name="c", subcore_axis_name="s")

    @functools.partial(
        pl.kernel, mesh=mesh,
        out_type=jax.ShapeDtypeStruct((PT,), jnp.float32),
        scratch_types=[pltpu.VMEM((CH,), jnp.int32),
                       pltpu.VMEM((CH,), jnp.float32),
                       pltpu.VMEM((CHL,), jnp.int32),
                       pltpu.VMEM((CHL,), jnp.float32),
                       pltpu.SemaphoreType.DMA],
    )
    def gk(tab_h, lin_h, out_h, iv, rv, ivt, rvt, sem):
        wid = lax.axis_index("s") * _NC + lax.axis_index("c")
        base = wid * CH

        @pl.when(wid < _NW - 1)
        def _main():
            pltpu.sync_copy(lin_h.at[pl.ds(base, CH)], iv)
            pltpu.async_copy(tab_h.at[iv], rv, sem).wait()
            pltpu.sync_copy(rv, out_h.at[pl.ds(base, CH)])

        @pl.when(wid == _NW - 1)
        def _tail():
            pltpu.sync_copy(lin_h.at[pl.ds(base, CHL)], ivt)
            pltpu.async_copy(tab_h.at[ivt], rvt, sem).wait()
            pltpu.sync_copy(rvt, out_h.at[pl.ds(base, CHL)])

    return gk(tab, lin)


def _sc_scatter(avals, rvals, linv, linr, zrow, GVt, GRt):
    PT = avals.shape[0]
    CH, CHL = _split(PT, _NS)
    SGV = GVt // _NS
    SGR = GRt // _NS
    mesh = plsc.VectorSubcoreMesh(core_axis_name="c", subcore_axis_name="s")

    @functools.partial(
        pl.kernel, mesh=mesh,
        out_type=[jax.ShapeDtypeStruct((GVt,), jnp.float32),
                  jax.ShapeDtypeStruct((GRt,), jnp.float32)],
        scratch_types=[pltpu.VMEM((CH,), jnp.int32),
                       pltpu.VMEM((CH,), jnp.float32),
                       pltpu.VMEM((CHL,), jnp.int32),
                       pltpu.VMEM((CHL,), jnp.float32),
                       pltpu.VMEM_SHARED((GVt,), jnp.float32),
                       pltpu.VMEM_SHARED((GRt,), jnp.float32),
                       pltpu.SemaphoreType.DMA],
    )
    def sk(av_h, ar_h, lv_h, lr_h, z_h, ov_h, or_h,
           iv, vv, ivt, vvt, accv, accr, sem):
        cid = lax.axis_index("c")
        sid = lax.axis_index("s")
        base = sid * CH

        @pl.when(cid == 0)
        def _zv():
            pltpu.sync_copy(z_h.at[pl.ds(0, SGV)],
                            accv.at[pl.ds(sid * SGV, SGV)])

        @pl.when(cid == 1)
        def _zr():
            pltpu.sync_copy(z_h.at[pl.ds(0, SGR)],
                            accr.at[pl.ds(sid * SGR, SGR)])

        plsc.subcore_barrier()

        @pl.when((cid == 0) & (sid < _NS - 1))
        def _v_main():
            pltpu.sync_copy(lv_h.at[pl.ds(base, CH)], iv)
            pltpu.sync_copy(av_h.at[pl.ds(base, CH)], vv)
            pltpu.sync_copy(vv, accv.at[iv], add=True)

        @pl.when((cid == 0) & (sid == _NS - 1))
        def _v_tail():
            pltpu.sync_copy(lv_h.at[pl.ds(base, CHL)], ivt)
            pltpu.sync_copy(av_h.at[pl.ds(base, CHL)], vvt)
            pltpu.sync_copy(vvt, accv.at[ivt], add=True)

        @pl.when((cid == 1) & (sid < _NS - 1))
        def _r_main():
            pltpu.sync_copy(lr_h.at[pl.ds(base, CH)], iv)
            pltpu.sync_copy(ar_h.at[pl.ds(base, CH)], vv)
            pltpu.sync_copy(vv, accr.at[iv], add=True)

        @pl.when((cid == 1) & (sid == _NS - 1))
        def _r_tail():
            pltpu.sync_copy(lr_h.at[pl.ds(base, CHL)], ivt)
            pltpu.sync_copy(ar_h.at[pl.ds(base, CHL)], vvt)
            pltpu.sync_copy(vvt, accr.at[ivt], add=True)

        plsc.subcore_barrier()

        @pl.when(cid == 0)
        def _pv():
            pltpu.sync_copy(accv.at[pl.ds(sid * SGV, SGV)],
                            ov_h.at[pl.ds(sid * SGV, SGV)])

        @pl.when(cid == 1)
        def _pr():
            pltpu.sync_copy(accr.at[pl.ds(sid * SGR, SGR)],
                            or_h.at[pl.ds(sid * SGR, SGR)])

    return sk(avals, rvals, linv, linr, zrow)


def kernel(p_feat, v_feat, r_feat, v2p_ind, r2p_ind,
           Wp, bp, Wv, bv, Wr, br, ap, av, ar):
    B, CP, N = p_feat.shape
    CV, HV, WVG = v_feat.shape[1:]
    CR, HR, WRG = r_feat.shape[1:]
    CE = Wp.shape[0]
    GV = HV * WVG
    GR = HR * WRG

    sv_grid = _score_grid(v_feat, Wv, av, bv, HB=64)
    sr_grid = _score_grid(r_feat, Wr, ar, br, HB=16)


    viT = jnp.transpose(v2p_ind, (0, 2, 1)).astype(jnp.int32)
    riT = jnp.transpose(r2p_ind, (0, 2, 1)).astype(jnp.int32)
    offv = (jnp.arange(B, dtype=jnp.int32) * GV)[:, None]
    offr = (jnp.arange(B, dtype=jnp.int32) * GR)[:, None]
    linv = (viT[:, 0, :] * WVG + viT[:, 1, :] + offv).reshape(-1)
    linr = (riT[:, 0, :] * WRG + riT[:, 1, :] + offr).reshape(-1)


    sv_pt = _sc_gather(sv_grid.reshape(-1), linv).reshape(B, 1, N)
    sr_pt = _sc_gather(sr_grid.reshape(-1), linr).reshape(B, 1, N)

    pT = jnp.transpose(p_feat, (0, 2, 1))
    xp, alpha_v, alpha_r = _point_kernel(pT, sv_pt, sr_pt, Wp, bp, ap,
                                         T=8192)

    zrow = jnp.zeros((B * GR // _NS,), jnp.float32)
    wv, wr = _sc_scatter(alpha_v.reshape(-1), alpha_r.reshape(-1),
                         linv, linr, zrow, B * GV, B * GR)

    xv = _map_kernel(v_feat, Wv, bv, wv, HB=64)
    xr = _map_kernel(r_feat, Wr, br, wr, HB=16)
    return (xp, xv, xr)


# speedup vs baseline: 1.0703x; 1.0223x over previous
import functools

import jax
import jax.numpy as jnp
from jax import lax
from jax.experimental import pallas as pl
from jax.experimental.pallas import tpu as pltpu
from jax.experimental.pallas import tpu_sc as plsc

_NC = 2
_NS = 16
_NW = _NC * _NS


def _split(total, nw=_NW):
    ch = ((total + nw - 1) // nw + 7) // 8 * 8
    last = total - (nw - 1) * ch
    assert last > 0 and last % 8 == 0 and ch % 8 == 0
    return ch, last


def _score_grid_body(f_ref, w_ref, a_ref, b_ref, s_ref):
    g = jnp.dot(a_ref[...], w_ref[...],
                preferred_element_type=jnp.float32)
    c0 = jnp.sum(a_ref[...] * b_ref[...])
    f = f_ref[0]
    s_ref[0, 0] = jnp.sum(f * g[0][:, None, None], axis=0) + c0


def _score_grid(feat, W, a, b, HB):
    Bn, C, H, Wg = feat.shape
    CE = W.shape[0]
    return pl.pallas_call(
        _score_grid_body,
        grid=(Bn, H // HB),
        in_specs=[
            pl.BlockSpec((1, C, HB, Wg), lambda bi, i: (bi, 0, i, 0)),
            pl.BlockSpec((CE, C), lambda bi, i: (0, 0)),
            pl.BlockSpec((1, CE), lambda bi, i: (0, 0)),
            pl.BlockSpec((1, CE), lambda bi, i: (0, 0)),
        ],
        out_specs=pl.BlockSpec((1, 1, HB, Wg), lambda bi, i: (bi, 0, i, 0)),
        out_shape=jax.ShapeDtypeStruct((Bn, 1, H, Wg), jnp.float32),
        compiler_params=pltpu.CompilerParams(
            dimension_semantics=("parallel", "parallel")),
    )(feat, W, a.reshape(1, CE), b.reshape(1, CE))


def _point_body(p_ref, sv_ref, sr_ref, w_ref, b_ref, a_ref,
                xp_ref, av_ref, ar_ref):
    emb = lax.dot_general(w_ref[...], p_ref[0],
                          (((1,), (1,)), ((), ())),
                          preferred_element_type=jnp.float32) + b_ref[...]
    sp = jnp.sum(emb * a_ref[...], axis=0, keepdims=True)
    sv = sv_ref[0]
    sr = sr_ref[0]
    m = jnp.maximum(sp, jnp.maximum(sv, sr))
    ep = jnp.exp(sp - m)
    ev = jnp.exp(sv - m)
    er = jnp.exp(sr - m)
    inv = 1.0 / (ep + ev + er)
    xp_ref[0] = emb * (ep * inv)
    av_ref[0] = ev * inv
    ar_ref[0] = er * inv


def _point_kernel(pT, sv_pt, sr_pt, Wp, bp, ap, T):
    Bn, N, C = pT.shape
    CE = Wp.shape[0]
    nt = pl.cdiv(N, T)
    return pl.pallas_call(
        _point_body,
        grid=(Bn, nt),
        in_specs=[
            pl.BlockSpec((1, T, C), lambda bi, i: (bi, i, 0)),
            pl.BlockSpec((1, 1, T), lambda bi, i: (bi, 0, i)),
            pl.BlockSpec((1, 1, T), lambda bi, i: (bi, 0, i)),
            pl.BlockSpec((CE, C), lambda bi, i: (0, 0)),
            pl.BlockSpec((CE, 1), lambda bi, i: (0, 0)),
            pl.BlockSpec((CE, 1), lambda bi, i: (0, 0)),
        ],
        out_specs=[
            pl.BlockSpec((1, CE, T), lambda bi, i: (bi, 0, i)),
            pl.BlockSpec((1, 1, T), lambda bi, i: (bi, 0, i)),
            pl.BlockSpec((1, 1, T), lambda bi, i: (bi, 0, i)),
        ],
        out_shape=[
            jax.ShapeDtypeStruct((Bn, CE, N), jnp.float32),
            jax.ShapeDtypeStruct((Bn, 1, N), jnp.float32),
            jax.ShapeDtypeStruct((Bn, 1, N), jnp.float32),
        ],
        compiler_params=pltpu.CompilerParams(
            dimension_semantics=("parallel", "parallel")),
    )(pT, sv_pt, sr_pt, Wp, bp.reshape(CE, 1), ap.reshape(CE, 1))


def _map_body(hb, wg, f_ref, w_ref, b_ref, wt_ref, o_ref):
    for h in range(hb):
        emb = jnp.dot(w_ref[...], f_ref[0, :, h, :],
                      preferred_element_type=jnp.float32) + b_ref[...]
        o_ref[0, :, h, :] = emb * wt_ref[pl.ds(h * wg, wg)][None, :]


def _map_kernel(feat, W, b, wt, HB):
    Bn, C, H, Wg = feat.shape
    CE = W.shape[0]
    nh = H // HB
    return pl.pallas_call(
        functools.partial(_map_body, HB, Wg),
        grid=(Bn, nh),
        in_specs=[
            pl.BlockSpec((1, C, HB, Wg), lambda bi, i: (bi, 0, i, 0)),
            pl.BlockSpec((CE, C), lambda bi, i: (0, 0)),
            pl.BlockSpec((CE, 1), lambda bi, i: (0, 0)),
            pl.BlockSpec((HB * Wg,), lambda bi, i, nh=nh: (bi * nh + i,)),
        ],
        out_specs=pl.BlockSpec((1, CE, HB, Wg), lambda bi, i: (bi, 0, i, 0)),
        out_shape=jax.ShapeDtypeStruct((Bn, CE, H, Wg), jnp.float32),
        compiler_params=pltpu.CompilerParams(
            dimension_semantics=("parallel", "parallel")),
    )(feat, W, b.reshape(CE, 1), wt)


def _sc_gather(tab, lin):
    PT = lin.shape[0]
    CH, CHL = _split(PT)
    mesh = plsc.VectorSubcoreMesh(core_axis_name="c", subcore_axis_name="s")

    @functools.partial(
        pl.kernel, mesh=mesh,
        out_type=jax.ShapeDtypeStruct((PT,), jnp.float32),
        scratch_types=[pltpu.VMEM((CH,), jnp.int32),
                       pltpu.VMEM((CH,), jnp.float32),
                       pltpu.VMEM((CHL,), jnp.int32),
                       pltpu.VMEM((CHL,), jnp.float32),
                       pltpu.SemaphoreType.DMA],
    )
    def gk(tab_h, lin_h, out_h, iv, rv, ivt, rvt, sem):
        wid = lax.axis_index("s") * _NC + lax.axis_index("c")
        base = wid * CH

        @pl.when(wid < _NW - 1)
        def _main():
            pltpu.sync_copy(lin_h.at[pl.ds(base, CH)], iv)
            pltpu.async_copy(tab_h.at[iv], rv, sem).wait()
            pltpu.sync_copy(rv, out_h.at[pl.ds(base, CH)])

        @pl.when(wid == _NW - 1)
        def _tail():
            pltpu.sync_copy(lin_h.at[pl.ds(base, CHL)], ivt)
            pltpu.async_copy(tab_h.at[ivt], rvt, sem).wait()
            pltpu.sync_copy(rvt, out_h.at[pl.ds(base, CHL)])

    return gk(tab, lin)


def _sc_scatter(avals, rvals, linv, linr, zrow, GVt, GRt):
    PT = avals.shape[0]
    CH, CHL = _split(PT, _NS)
    SGV = GVt // _NS
    SGR = GRt // _NS
    mesh = plsc.VectorSubcoreMesh(core_axis_name="c", subcore_axis_name="s")

    @functools.partial(
        pl.kernel, mesh=mesh,
        out_type=[jax.ShapeDtypeStruct((GVt,), jnp.float32),
                  jax.ShapeDtypeStruct((GRt,), jnp.float32)],
        scratch_types=[pltpu.VMEM((CH,), jnp.int32),
                       pltpu.VMEM((CH,), jnp.float32),
                       pltpu.VMEM((CHL,), jnp.int32),
                       pltpu.VMEM((CHL,), jnp.float32),
                       pltpu.VMEM_SHARED((GVt,), jnp.float32),
                       pltpu.VMEM_SHARED((GRt,), jnp.float32),
                       pltpu.SemaphoreType.DMA],
    )
    def sk(av_h, ar_h, lv_h, lr_h, z_h, ov_h, or_h,
           iv, vv, ivt, vvt, accv, accr, sem):
        cid = lax.axis_index("c")
        sid = lax.axis_index("s")
        base = sid * CH

        @pl.when(cid == 0)
        def _zv():
            pltpu.sync_copy(z_h.at[pl.ds(0, SGV)],
                            accv.at[pl.ds(sid * SGV, SGV)])

        @pl.when(cid == 1)
        def _zr():
            pltpu.sync_copy(z_h.at[pl.ds(0, SGR)],
                            accr.at[pl.ds(sid * SGR, SGR)])

        plsc.subcore_barrier()

        @pl.when((cid == 0) & (sid < _NS - 1))
        def _v_main():
            pltpu.sync_copy(lv_h.at[pl.ds(base, CH)], iv)
            pltpu.sync_copy(av_h.at[pl.ds(base, CH)], vv)
            pltpu.sync_copy(vv, accv.at[iv], add=True)

        @pl.when((cid == 0) & (sid == _NS - 1))
        def _v_tail():
            pltpu.sync_copy(lv_h.at[pl.ds(base, CHL)], ivt)
            pltpu.sync_copy(av_h.at[pl.ds(base, CHL)], vvt)
            pltpu.sync_copy(vvt, accv.at[ivt], add=True)

        @pl.when((cid == 1) & (sid < _NS - 1))
        def _r_main():
            pltpu.sync_copy(lr_h.at[pl.ds(base, CH)], iv)
            pltpu.sync_copy(ar_h.at[pl.ds(base, CH)], vv)
            pltpu.sync_copy(vv, accr.at[iv], add=True)

        @pl.when((cid == 1) & (sid == _NS - 1))
        def _r_tail():
            pltpu.sync_copy(lr_h.at[pl.ds(base, CHL)], ivt)
            pltpu.sync_copy(ar_h.at[pl.ds(base, CHL)], vvt)
            pltpu.sync_copy(vvt, accr.at[ivt], add=True)

        plsc.subcore_barrier()

        @pl.when(cid == 0)
        def _pv():
            pltpu.sync_copy(accv.at[pl.ds(sid * SGV, SGV)],
                            ov_h.at[pl.ds(sid * SGV, SGV)])

        @pl.when(cid == 1)
        def _pr():
            pltpu.sync_copy(accr.at[pl.ds(sid * SGR, SGR)],
                            or_h.at[pl.ds(sid * SGR, SGR)])

    return sk(avals, rvals, linv, linr, zrow)


def kernel(p_feat, v_feat, r_feat, v2p_ind, r2p_ind,
           Wp, bp, Wv, bv, Wr, br, ap, av, ar):
    B, CP, N = p_feat.shape
    CV, HV, WVG = v_feat.shape[1:]
    CR, HR, WRG = r_feat.shape[1:]
    CE = Wp.shape[0]
    GV = HV * WVG
    GR = HR * WRG

    sv_grid = _score_grid(v_feat, Wv, av, bv, HB=64)
    sr_grid = _score_grid(r_feat, Wr, ar, br, HB=32)


    viT = jnp.transpose(v2p_ind, (0, 2, 1)).astype(jnp.int32)
    riT = jnp.transpose(r2p_ind, (0, 2, 1)).astype(jnp.int32)
    offv = (jnp.arange(B, dtype=jnp.int32) * GV)[:, None]
    offr = (jnp.arange(B, dtype=jnp.int32) * GR)[:, None]
    linv = (viT[:, 0, :] * WVG + viT[:, 1, :] + offv).reshape(-1)
    linr = (riT[:, 0, :] * WRG + riT[:, 1, :] + offr).reshape(-1)


    sv_pt = _sc_gather(sv_grid.reshape(-1), linv).reshape(B, 1, N)
    sr_pt = _sc_gather(sr_grid.reshape(-1), linr).reshape(B, 1, N)

    pT = jnp.transpose(p_feat, (0, 2, 1))
    xp, alpha_v, alpha_r = _point_kernel(pT, sv_pt, sr_pt, Wp, bp, ap,
                                         T=12544)

    zrow = jnp.zeros((B * GR // _NS,), jnp.float32)
    wv, wr = _sc_scatter(alpha_v.reshape(-1), alpha_r.reshape(-1),
                         linv, linr, zrow, B * GV, B * GR)

    xv = _map_kernel(v_feat, Wv, bv, wv, HB=64)
    xr = _map_kernel(r_feat, Wr, br, wr, HB=16)
    return (xp, xv, xr)
